# initial kernel scaffold (unmeasured)
import jax
import jax.numpy as jnp
from jax import lax
from jax.experimental import pallas as pl
from jax.experimental.pallas import tpu as pltpu


def kernel(
    x,
):
    def body(*refs):
        pass

    out_shape = jax.ShapeDtypeStruct(..., jnp.float32)
    return pl.pallas_call(body, out_shape=out_shape)(...)


# baseline (device time: 1074148 ns/iter reference)
import jax
import jax.numpy as jnp
from jax import lax
from jax.experimental import pallas as pl
from jax.experimental.pallas import tpu as pltpu

N_CHUNKS = 8


def kernel(x):
    m_per, n = x.shape
    rows = m_per // N_CHUNKS

    def body(x_ref, out_ref, send_sems, recv_sems, copy_sem):
        my_x = lax.axis_index("x")
        my_y = lax.axis_index("y")
        my_z = lax.axis_index("z")

        rdmas = []
        for c in range(N_CHUNKS):
            rdma = pltpu.make_async_remote_copy(
                src_ref=x_ref.at[pl.ds(c * rows, rows), :],
                dst_ref=out_ref.at[pl.ds(my_x * m_per + c * rows, rows), :],
                send_sem=send_sems.at[c],
                recv_sem=recv_sems.at[c],
                device_id=(1 - my_x, my_y, my_z),
                device_id_type=pl.DeviceIdType.MESH,
            )
            rdma.start()
            rdmas.append(rdma)

        local = pltpu.make_async_copy(
            x_ref,
            out_ref.at[pl.ds(my_x * m_per, m_per), :],
            copy_sem,
        )
        local.start()
        local.wait()

        for rdma in rdmas:
            rdma.wait()

    out_shape = jax.ShapeDtypeStruct((2 * m_per, n), x.dtype)
    return pl.pallas_call(
        body,
        out_shape=out_shape,
        in_specs=[pl.BlockSpec(memory_space=pltpu.MemorySpace.HBM)],
        out_specs=pl.BlockSpec(memory_space=pltpu.MemorySpace.HBM),
        scratch_shapes=[
            pltpu.SemaphoreType.DMA((N_CHUNKS,)),
            pltpu.SemaphoreType.DMA((N_CHUNKS,)),
            pltpu.SemaphoreType.DMA,
        ],
    )(x)
